# baseline (speedup 1.0000x reference)
import jax
import jax.numpy as jnp
from jax.experimental import pallas as pl
from jax.experimental.pallas import tpu as pltpu

N_ROWS = 8192
NUM_FEATS = 4096
ROW_BLOCK = 256
LANES = 128
N_BLOCKS = NUM_FEATS // LANES


def _gather_body(x_ref, idx_ref, o_ref):
    idx = idx_ref[...]
    lanes = jax.lax.rem(idx, LANES)
    blks = jax.lax.div(idx, LANES)
    for c in range(N_BLOCKS):
        sl = slice(c * LANES, (c + 1) * LANES)
        lane_c = jnp.broadcast_to(lanes[:, sl], (ROW_BLOCK, LANES))
        blk_c = blks[:, sl]
        acc = jnp.zeros((ROW_BLOCK, LANES), jnp.float32)
        for s in range(N_BLOCKS):
            g = jnp.take_along_axis(
                x_ref[:, s * LANES:(s + 1) * LANES], lane_c, axis=1)
            acc = jnp.where(blk_c == s, g, acc)
        o_ref[:, sl] = acc


def kernel(x, indices):
    indices = indices.astype(jnp.int32).reshape(1, NUM_FEATS)
    grid = (N_ROWS // ROW_BLOCK,)
    return pl.pallas_call(
        _gather_body,
        grid=grid,
        in_specs=[
            pl.BlockSpec((ROW_BLOCK, NUM_FEATS), lambda i: (i, 0)),
            pl.BlockSpec((1, NUM_FEATS), lambda i: (0, 0)),
        ],
        out_specs=pl.BlockSpec((ROW_BLOCK, NUM_FEATS), lambda i: (i, 0)),
        out_shape=jax.ShapeDtypeStruct((N_ROWS, NUM_FEATS), x.dtype),
        compiler_params=pltpu.CompilerParams(
            dimension_semantics=("parallel",),
        ),
    )(x, indices)


# speedup vs baseline: 2.2180x; 2.2180x over previous
import jax
import jax.numpy as jnp
from jax.experimental import pallas as pl
from jax.experimental.pallas import tpu as pltpu
from jax.experimental.pallas import tpu_sc as plsc

N_ROWS = 8192
NUM_FEATS = 4096

CHUNK = 256
N_CHUNKS = N_ROWS // CHUNK
JB = 512
TABLE_ROWS = N_CHUNKS * NUM_FEATS

GATHER_WINDOW = 128


def _transpose_body(x_ref, o_ref):
    o_ref[...] = x_ref[...].T


def _transpose_fwd(x):
    grid = (N_CHUNKS, NUM_FEATS // JB)
    return pl.pallas_call(
        _transpose_body,
        grid=grid,
        in_specs=[pl.BlockSpec((CHUNK, JB), lambda k, jb: (k, jb))],
        out_specs=pl.BlockSpec(
            (JB, CHUNK), lambda k, jb: (k * (NUM_FEATS // JB) + jb, 0)),
        out_shape=jax.ShapeDtypeStruct((TABLE_ROWS, CHUNK), x.dtype),
        compiler_params=pltpu.CompilerParams(
            dimension_semantics=("arbitrary", "arbitrary"),
        ),
    )(x)


def _transpose_bwd(g):
    grid = (N_CHUNKS, NUM_FEATS // JB)
    return pl.pallas_call(
        _transpose_body,
        grid=grid,
        in_specs=[pl.BlockSpec(
            (JB, CHUNK), lambda k, jb: (k * (NUM_FEATS // JB) + jb, 0))],
        out_specs=pl.BlockSpec((CHUNK, JB), lambda k, jb: (k, jb)),
        out_shape=jax.ShapeDtypeStruct((N_ROWS, NUM_FEATS), g.dtype),
        compiler_params=pltpu.CompilerParams(
            dimension_semantics=("arbitrary", "arbitrary"),
        ),
    )(g)


def _sc_gather(table, idx3):
    vector_mesh = plsc.VectorSubcoreMesh(
        core_axis_name="core", subcore_axis_name="subcore")
    n_windows = TABLE_ROWS // GATHER_WINDOW

    @pl.kernel(out_type=jax.ShapeDtypeStruct(table.shape, table.dtype),
               mesh=vector_mesh)
    def k(x_hbm, i_hbm, o_hbm):
        def body(i_vmem, o_vmem):
            pltpu.sync_copy(x_hbm.at[i_vmem.at[0, 0]], o_vmem)

        pltpu.emit_pipeline(
            body,
            grid=(n_windows,),
            in_specs=[pl.BlockSpec((1, 1, GATHER_WINDOW),
                                   index_map=lambda i: (i, 0, 0))],
            out_specs=[pl.BlockSpec((GATHER_WINDOW, CHUNK),
                                    index_map=lambda i: (i, 0))],
            core_axis_name=("core", "subcore"),
            dimension_semantics=(pltpu.PARALLEL,),
        )(i_hbm, o_hbm)

    return k(table, idx3)


def kernel(x, indices):
    idx = indices.astype(jnp.int32)
    idx3 = (jnp.arange(N_CHUNKS, dtype=jnp.int32)[:, None] * NUM_FEATS
            + idx[None, :]).reshape(TABLE_ROWS // GATHER_WINDOW, 1,
                                    GATHER_WINDOW)
    xt = _transpose_fwd(x)
    gt = _sc_gather(xt, idx3)
    return _transpose_bwd(gt)
